# baseline (speedup 1.0000x reference)
import functools

import jax
import jax.numpy as jnp
from jax.experimental import pallas as pl
from jax.experimental.pallas import tpu as pltpu

_TILE = 2048
_NCORES = 2


def _phase_a(e_ref, w1_ref, b1_ref, w2_ref, st_ref, en_ref,
             accp_ref, mp_ref, zp_ref, acc_ref, m_ref, z_ref, *, tile, batch):
    j = pl.program_id(1)

    @pl.when(j == 0)
    def _init():
        m_ref[0] = -jnp.inf
        z_ref[0] = 0.0
        acc_ref[...] = jnp.zeros_like(acc_ref)

    c = pl.program_id(0)
    steps = pl.num_programs(1)
    e = e_ref[...]
    h = jax.lax.dot_general(
        e, w1_ref[...], (((1,), (1,)), ((), ())),
        preferred_element_type=jnp.float32)
    h = jnp.maximum(h + b1_ref[...], 0.0)
    logit = jax.lax.dot_general(
        h, w2_ref[...], (((1,), (1,)), ((), ())),
        preferred_element_type=jnp.float32)

    m_old = m_ref[0]
    m_new = jnp.maximum(m_old, jnp.max(logit))
    alpha = jnp.exp(m_old - m_new)
    s = jnp.exp(logit - m_new)
    z_ref[0] = z_ref[0] * alpha + jnp.sum(s)

    rows = (jax.lax.broadcasted_iota(jnp.int32, (tile, batch), 0)
            + (c * steps + j) * tile)
    mask = jnp.logical_and(rows >= st_ref[...], rows < en_ref[...])
    masked = jnp.where(mask, s, 0.0)
    contrib = jax.lax.dot_general(
        masked, e, (((0,), (0,)), ((), ())),
        preferred_element_type=jnp.float32)
    acc_ref[...] = acc_ref[...] * alpha + contrib
    m_ref[0] = m_new

    @pl.when(j == steps - 1)
    def _finish():
        accp_ref[0] = acc_ref[...]
        mp_ref[0, 0, 0] = m_ref[0]
        zp_ref[0, 0, 0] = z_ref[0]


def _phase_b(accp_ref, mp_ref, zp_ref, out_ref, *, ncores):
    m = mp_ref[0, 0, 0]
    for c in range(1, ncores):
        m = jnp.maximum(m, mp_ref[c, 0, 0])
    z = zp_ref[0, 0, 0] * jnp.exp(mp_ref[0, 0, 0] - m)
    for c in range(1, ncores):
        z = z + zp_ref[c, 0, 0] * jnp.exp(mp_ref[c, 0, 0] - m)
    out = accp_ref[0] * (jnp.exp(mp_ref[0, 0, 0] - m) / z)
    for c in range(1, ncores):
        out = out + accp_ref[c] * (jnp.exp(mp_ref[c, 0, 0] - m) / z)
    out_ref[...] = out


def kernel(embeddings, lengths, W1, b1, W2, b2):
    total, embed_dim = embeddings.shape
    batch = lengths.shape[0]
    tile = _TILE
    steps = total // (tile * _NCORES)

    ends = jnp.cumsum(lengths.astype(jnp.int32))
    starts = ends - lengths
    st = starts.reshape(1, batch)
    en = ends.reshape(1, batch)
    b1r = b1.reshape(1, embed_dim)

    accp, mp, zp = pl.pallas_call(
        functools.partial(_phase_a, tile=tile, batch=batch),
        grid=(_NCORES, steps),
        in_specs=[
            pl.BlockSpec((tile, embed_dim),
                         lambda c, j, s=steps: (c * s + j, 0)),
            pl.BlockSpec((embed_dim, embed_dim), lambda c, j: (0, 0)),
            pl.BlockSpec((1, embed_dim), lambda c, j: (0, 0)),
            pl.BlockSpec((1, embed_dim), lambda c, j: (0, 0)),
            pl.BlockSpec((1, batch), lambda c, j: (0, 0)),
            pl.BlockSpec((1, batch), lambda c, j: (0, 0)),
        ],
        out_specs=[
            pl.BlockSpec((1, batch, embed_dim), lambda c, j: (c, 0, 0)),
            pl.BlockSpec(memory_space=pltpu.SMEM, block_shape=(1, 1, 1),
                         index_map=lambda c, j: (c, 0, 0)),
            pl.BlockSpec(memory_space=pltpu.SMEM, block_shape=(1, 1, 1),
                         index_map=lambda c, j: (c, 0, 0)),
        ],
        out_shape=[
            jax.ShapeDtypeStruct((_NCORES, batch, embed_dim), jnp.float32),
            jax.ShapeDtypeStruct((_NCORES, 1, 1), jnp.float32),
            jax.ShapeDtypeStruct((_NCORES, 1, 1), jnp.float32),
        ],
        scratch_shapes=[
            pltpu.VMEM((batch, embed_dim), jnp.float32),
            pltpu.SMEM((1,), jnp.float32),
            pltpu.SMEM((1,), jnp.float32),
        ],
        compiler_params=pltpu.CompilerParams(
            dimension_semantics=("parallel", "arbitrary"),
        ),
    )(embeddings, W1, b1r, W2, st, en)

    out = pl.pallas_call(
        functools.partial(_phase_b, ncores=_NCORES),
        in_specs=[
            pl.BlockSpec((_NCORES, batch, embed_dim), lambda: (0, 0, 0)),
            pl.BlockSpec(memory_space=pltpu.SMEM),
            pl.BlockSpec(memory_space=pltpu.SMEM),
        ],
        out_specs=pl.BlockSpec((batch, embed_dim), lambda: (0, 0)),
        out_shape=jax.ShapeDtypeStruct((batch, embed_dim), jnp.float32),
    )(accp, mp, zp)
    return out


# speedup vs baseline: 1.1106x; 1.1106x over previous
import functools

import jax
import jax.numpy as jnp
from jax.experimental import pallas as pl
from jax.experimental.pallas import tpu as pltpu

_TILE = 4096


def _dwa_kernel(e_ref, w1_ref, b1_ref, w2_ref, st_ref, en_ref, out_ref,
                acc_ref, m_ref, z_ref, *, tile, batch):
    i = pl.program_id(0)

    @pl.when(i == 0)
    def _init():
        m_ref[0] = -jnp.inf
        z_ref[0] = 0.0
        acc_ref[...] = jnp.zeros_like(acc_ref)

    e = e_ref[...]
    h = jax.lax.dot_general(
        e, w1_ref[...], (((1,), (1,)), ((), ())),
        preferred_element_type=jnp.float32)
    h = jnp.maximum(h + b1_ref[...], 0.0)
    logit = jax.lax.dot_general(
        h, w2_ref[...], (((1,), (1,)), ((), ())),
        preferred_element_type=jnp.float32)

    m_old = m_ref[0]
    m_new = jnp.maximum(m_old, jnp.max(logit))
    alpha = jnp.exp(m_old - m_new)
    s = jnp.exp(logit - m_new)
    z_ref[0] = z_ref[0] * alpha + jnp.sum(s)

    st = st_ref[...]
    en = en_ref[...]
    rows = jax.lax.broadcasted_iota(jnp.int32, (tile, batch), 0) + i * tile
    mask = jnp.logical_and(rows >= st, rows < en)
    masked = jnp.where(mask, s, 0.0)
    contrib = jax.lax.dot_general(
        masked, e, (((0,), (0,)), ((), ())),
        preferred_element_type=jnp.float32)
    acc_ref[...] = acc_ref[...] * alpha + contrib
    m_ref[0] = m_new

    @pl.when(i == pl.num_programs(0) - 1)
    def _finish():
        out_ref[...] = acc_ref[...] / z_ref[0]


def kernel(embeddings, lengths, W1, b1, W2, b2):
    total, embed_dim = embeddings.shape
    batch = lengths.shape[0]
    tile = _TILE
    num_tiles = total // tile

    out_call = pl.pallas_call(
        functools.partial(_dwa_kernel, tile=tile, batch=batch),
        grid=(num_tiles,),
        in_specs=[
            pl.BlockSpec((tile, embed_dim), lambda i: (i, 0)),
            pl.BlockSpec((embed_dim, embed_dim), lambda i: (0, 0)),
            pl.BlockSpec((1, embed_dim), lambda i: (0, 0)),
            pl.BlockSpec((1, embed_dim), lambda i: (0, 0)),
            pl.BlockSpec((1, batch), lambda i: (0, 0)),
            pl.BlockSpec((1, batch), lambda i: (0, 0)),
        ],
        out_specs=pl.BlockSpec((batch, embed_dim), lambda i: (0, 0)),
        out_shape=jax.ShapeDtypeStruct((batch, embed_dim), jnp.float32),
        scratch_shapes=[
            pltpu.VMEM((batch, embed_dim), jnp.float32),
            pltpu.SMEM((1,), jnp.float32),
            pltpu.SMEM((1,), jnp.float32),
        ],
        compiler_params=pltpu.CompilerParams(
            dimension_semantics=("arbitrary",),
        ),
    )
    ends = jnp.cumsum(lengths.astype(jnp.int32))
    starts = ends - lengths
    out = out_call(embeddings, W1, b1.reshape(1, embed_dim), W2,
                   starts.reshape(1, batch), ends.reshape(1, batch))
    return out
